# Initial kernel scaffold; baseline (speedup 1.0000x reference)
import jax
import jax.numpy as jnp
from jax.experimental import pallas as pl


def kernel(x, lut_weights):
    raise NotImplementedError("write your pallas kernel here")


# speedup vs baseline: 233.8520x; 233.8520x over previous
import functools

import numpy as np
import jax
import jax.numpy as jnp
from jax import lax
from jax.experimental import pallas as pl
from jax.experimental.pallas import tpu as pltpu
from jax.experimental.pallas import tpu_sc as plsc

_TUPLE = 4
_NIN = 4096
_ENT = 16
_BATCH = 1024
_L = 16
_NW = 32
_RPW = _BATCH // _NW
_HALF = 16
_VPR = _NIN // _L

_GATHER_DNUMS = lax.GatherDimensionNumbers(
    offset_dims=(), collapsed_slice_dims=(0,), start_index_map=(0,))


def _lane_shuffle(v, idx_const):
    return lax.gather(v, idx_const[:, None],
                      dimension_numbers=_GATHER_DNUMS, slice_sizes=(1,),
                      mode=lax.GatherScatterMode.PROMISE_IN_BOUNDS)


def _body(x_hbm, w_hbm, out_hbm, xbuf, wbuf, resbuf, outbuf):
    nc = 2
    wid = lax.axis_index("s") * nc + lax.axis_index("c")
    base = wid * _RPW

    lane_iota = lax.iota(jnp.int32, _L)
    pat = (jnp.int32(1) << (lane_iota & 3)).astype(jnp.float32)
    rot1 = (lane_iota + 1) & (_L - 1)
    rot2 = (lane_iota + 2) & (_L - 1)

    pltpu.sync_copy(w_hbm.at[pl.ds(base, _RPW)], wbuf)

    for h in range(_RPW // _HALF):
        pltpu.sync_copy(x_hbm.at[pl.ds(base + h * _HALF, _HALF)], xbuf)

        def row_step(r, _):
            wrow_idx = jnp.broadcast_to(h * _HALF + r, (_L,)).astype(jnp.int32)

            def vstep(i, acc):
                v = xbuf[r, pl.ds(i * _L, _L)]
                t = v * pat
                s = t + _lane_shuffle(t, rot1)
                s = s + _lane_shuffle(s, rot2)
                idx = s.astype(jnp.int32)
                return acc + plsc.load_gather(wbuf, [wrow_idx, idx])

            acc = lax.fori_loop(0, _VPR, vstep, jnp.zeros((_L,), jnp.float32))
            resbuf[r, :] = acc
            return _

        lax.fori_loop(0, _HALF, row_step, 0)

        tot = jnp.zeros((_L,), jnp.float32)
        for c in range(0, _L, _TUPLE):
            col = jnp.full((_L,), c, jnp.int32)
            tot = tot + plsc.load_gather(resbuf, [lane_iota, col])
        outbuf[pl.ds(h * _HALF, _HALF)] = tot

    pltpu.sync_copy(outbuf, out_hbm.at[pl.ds(base, _RPW)])


@jax.jit
def kernel(x, lut_weights):
    mesh = plsc.VectorSubcoreMesh(core_axis_name="c", subcore_axis_name="s")
    run = pl.kernel(
        _body,
        out_type=jax.ShapeDtypeStruct((_BATCH,), jnp.float32),
        mesh=mesh,
        compiler_params=pltpu.CompilerParams(needs_layout_passes=False),
        scratch_types=[
            pltpu.VMEM((_HALF, _NIN), jnp.float32),
            pltpu.VMEM((_RPW, _ENT), jnp.float32),
            pltpu.VMEM((_HALF, _ENT), jnp.float32),
            pltpu.VMEM((_RPW,), jnp.float32),
        ],
    )
    return run(x, lut_weights)


# speedup vs baseline: 328.5994x; 1.4052x over previous
import functools

import numpy as np
import jax
import jax.numpy as jnp
from jax import lax
from jax.experimental import pallas as pl
from jax.experimental.pallas import tpu as pltpu
from jax.experimental.pallas import tpu_sc as plsc

_TUPLE = 4
_NIN = 4096
_ENT = 16
_BATCH = 1024
_L = 16
_NW = 32
_RPW = _BATCH // _NW
_HALF = 16
_VPR = _NIN // _L

_GATHER_DNUMS = lax.GatherDimensionNumbers(
    offset_dims=(), collapsed_slice_dims=(0,), start_index_map=(0,))


def _lane_shuffle(v, idx_const):
    return lax.gather(v, idx_const[:, None],
                      dimension_numbers=_GATHER_DNUMS, slice_sizes=(1,),
                      mode=lax.GatherScatterMode.PROMISE_IN_BOUNDS)


_CHUNK = 8
_NCHUNK = _RPW // _CHUNK
_UNROLL = 8


def _body(x_hbm, w_hbm, out_hbm, xbuf, wbuf, resbuf, outbuf, sem0, sem1):
    nc = 2
    wid = lax.axis_index("s") * nc + lax.axis_index("c")
    base = wid * _RPW

    lane_iota = lax.iota(jnp.int32, _L)
    pat = (jnp.int32(1) << (lane_iota & 3)).astype(jnp.float32)
    rot1 = (lane_iota + 1) & (_L - 1)
    rot2 = (lane_iota + 2) & (_L - 1)

    pltpu.sync_copy(w_hbm.at[pl.ds(base, _RPW)], wbuf)

    sems = [sem0, sem1]
    copies = [None, None]
    copies[0] = pltpu.async_copy(
        x_hbm.at[pl.ds(base, _CHUNK)], xbuf.at[0], sems[0])

    for ch in range(_NCHUNK):
        slot = ch % 2
        if ch + 1 < _NCHUNK:
            nslot = (ch + 1) % 2
            copies[nslot] = pltpu.async_copy(
                x_hbm.at[pl.ds(base + (ch + 1) * _CHUNK, _CHUNK)],
                xbuf.at[nslot], sems[nslot])
        copies[slot].wait()

        def row_step(r, _):
            wrow_idx = jnp.broadcast_to(ch * _CHUNK + r, (_L,)).astype(jnp.int32)

            def vstep(k, acc):
                for j in range(_UNROLL):
                    v = xbuf[slot, r, pl.ds((k * _UNROLL + j) * _L, _L)]
                    t = v * pat
                    s = t + _lane_shuffle(t, rot1)
                    s = s + _lane_shuffle(s, rot2)
                    idx = s.astype(jnp.int32)
                    acc = acc + plsc.load_gather(wbuf, [wrow_idx, idx])
                return acc

            acc = lax.fori_loop(0, _VPR // _UNROLL, vstep,
                                jnp.zeros((_L,), jnp.float32))
            resbuf[ch * _CHUNK + r, :] = acc
            return _

        lax.fori_loop(0, _CHUNK, row_step, 0)

    for h in range(_RPW // _L):
        rows = lane_iota + h * _L
        tot = jnp.zeros((_L,), jnp.float32)
        for c in range(0, _L, _TUPLE):
            col = jnp.full((_L,), c, jnp.int32)
            tot = tot + plsc.load_gather(resbuf, [rows, col])
        outbuf[pl.ds(h * _L, _L)] = tot

    pltpu.sync_copy(outbuf, out_hbm.at[pl.ds(base, _RPW)])


@jax.jit
def kernel(x, lut_weights):
    mesh = plsc.VectorSubcoreMesh(core_axis_name="c", subcore_axis_name="s")
    run = pl.kernel(
        _body,
        out_type=jax.ShapeDtypeStruct((_BATCH,), jnp.float32),
        mesh=mesh,
        compiler_params=pltpu.CompilerParams(needs_layout_passes=False),
        scratch_types=[
            pltpu.VMEM((2, _CHUNK, _NIN), jnp.float32),
            pltpu.VMEM((_RPW, _ENT), jnp.float32),
            pltpu.VMEM((_RPW, _ENT), jnp.float32),
            pltpu.VMEM((_RPW,), jnp.float32),
            pltpu.SemaphoreType.DMA,
            pltpu.SemaphoreType.DMA,
        ],
    )
    return run(x, lut_weights)


# speedup vs baseline: 343.5499x; 1.0455x over previous
import functools

import numpy as np
import jax
import jax.numpy as jnp
from jax import lax
from jax.experimental import pallas as pl
from jax.experimental.pallas import tpu as pltpu
from jax.experimental.pallas import tpu_sc as plsc

_TUPLE = 4
_NIN = 4096
_ENT = 16
_BATCH = 1024
_L = 16
_NW = 32
_RPW = _BATCH // _NW
_HALF = 16
_VPR = _NIN // _L

_GATHER_DNUMS = lax.GatherDimensionNumbers(
    offset_dims=(), collapsed_slice_dims=(0,), start_index_map=(0,))


def _lane_shuffle(v, idx_const):
    return lax.gather(v, idx_const[:, None],
                      dimension_numbers=_GATHER_DNUMS, slice_sizes=(1,),
                      mode=lax.GatherScatterMode.PROMISE_IN_BOUNDS)


_CHUNK = 8
_NCHUNK = _RPW // _CHUNK
_UNROLL = 8


def _body(x_hbm, w_hbm, out_hbm, xbuf, wbuf, resbuf, outbuf, sem0, sem1):
    nc = 2
    wid = lax.axis_index("s") * nc + lax.axis_index("c")
    base = wid * _RPW

    lane_iota = lax.iota(jnp.int32, _L)
    pat = (jnp.int32(1) << (lane_iota & 3)).astype(jnp.float32)
    rot1 = (lane_iota + 1) & (_L - 1)
    rot2 = (lane_iota + 2) & (_L - 1)

    pltpu.sync_copy(w_hbm.at[pl.ds(base, _RPW)], wbuf)

    sems = [sem0, sem1]
    copies = [None, None]
    copies[0] = pltpu.async_copy(
        x_hbm.at[pl.ds(base, _CHUNK)], xbuf.at[0], sems[0])

    for ch in range(_NCHUNK):
        slot = ch % 2
        if ch + 1 < _NCHUNK:
            nslot = (ch + 1) % 2
            copies[nslot] = pltpu.async_copy(
                x_hbm.at[pl.ds(base + (ch + 1) * _CHUNK, _CHUNK)],
                xbuf.at[nslot], sems[nslot])
        copies[slot].wait()

        @plsc.parallel_loop(0, _CHUNK)
        def row_step(r):
            wrow_idx = jnp.broadcast_to(ch * _CHUNK + r, (_L,)).astype(jnp.int32)
            zero = jnp.zeros((_L,), jnp.float32)

            @plsc.parallel_loop(0, _VPR, unroll=_UNROLL, carry=(zero, zero))
            def accs(i, acc):
                a0, a1 = acc
                v = xbuf[slot, r, pl.ds(i * _L, _L)]
                t = v * pat
                s = t + _lane_shuffle(t, rot1)
                s = s + _lane_shuffle(s, rot2)
                idx = s.astype(jnp.int32)
                return (a1, a0 + plsc.load_gather(wbuf, [wrow_idx, idx]))

            resbuf[ch * _CHUNK + r, :] = accs[0] + accs[1]

    for h in range(_RPW // _L):
        rows = lane_iota + h * _L
        tot = jnp.zeros((_L,), jnp.float32)
        for c in range(0, _L, _TUPLE):
            col = jnp.full((_L,), c, jnp.int32)
            tot = tot + plsc.load_gather(resbuf, [rows, col])
        outbuf[pl.ds(h * _L, _L)] = tot

    pltpu.sync_copy(outbuf, out_hbm.at[pl.ds(base, _RPW)])


@jax.jit
def kernel(x, lut_weights):
    mesh = plsc.VectorSubcoreMesh(core_axis_name="c", subcore_axis_name="s")
    run = pl.kernel(
        _body,
        out_type=jax.ShapeDtypeStruct((_BATCH,), jnp.float32),
        mesh=mesh,
        compiler_params=pltpu.CompilerParams(needs_layout_passes=False),
        scratch_types=[
            pltpu.VMEM((2, _CHUNK, _NIN), jnp.float32),
            pltpu.VMEM((_RPW, _ENT), jnp.float32),
            pltpu.VMEM((_RPW, _ENT), jnp.float32),
            pltpu.VMEM((_RPW,), jnp.float32),
            pltpu.SemaphoreType.DMA,
            pltpu.SemaphoreType.DMA,
        ],
    )
    return run(x, lut_weights)
